# baseline (speedup 1.0000x reference)
import functools

import jax
import jax.numpy as jnp
from jax import lax
from jax.experimental import pallas as pl
from jax.experimental.pallas import tpu as pltpu
from jax.experimental.pallas import tpu_sc as plsc

_TOP_K = 8
_BT = 512
_CHUNKS = (2048, 4096, 5120, 5120)
_L = 16


def _matmul_body(x_ref, w_ref, logits_ref):
    logits_ref[...] = jax.lax.dot_general(
        x_ref[...], w_ref[...], (((1,), (1,)), ((), ())),
        preferred_element_type=jnp.float32,
    )


def _tc_logits(x, w, base_tokens, n_chunk_tokens):
    hidden_dim = x.shape[1]
    n_experts = w.shape[0]
    base = base_tokens // _BT
    return pl.pallas_call(
        _matmul_body,
        grid=(n_chunk_tokens // _BT,),
        in_specs=[
            pl.BlockSpec((_BT, hidden_dim), lambda i: (base + i, 0)),
            pl.BlockSpec((n_experts, hidden_dim), lambda i: (0, 0)),
        ],
        out_specs=pl.BlockSpec((_BT, n_experts), lambda i: (i, 0)),
        out_shape=jax.ShapeDtypeStruct((n_chunk_tokens, n_experts), jnp.float32),
    )(x, w)


def _sc_topk(logits, n_rows):
    nc, ns = 2, 16
    nw = nc * ns
    rpw = n_rows // nw

    gdn = lax.GatherDimensionNumbers(
        offset_dims=(), collapsed_slice_dims=(0,), start_index_map=(0,)
    )

    def _shuf(x, idx):
        return lax.gather(
            x, idx[:, None], gdn, (1,),
            mode=lax.GatherScatterMode.PROMISE_IN_BOUNDS,
        )

    mesh = plsc.VectorSubcoreMesh(
        core_axis_name="c", subcore_axis_name="s", num_cores=nc, num_subcores=ns
    )

    @functools.partial(
        pl.kernel,
        out_type=[
            jax.ShapeDtypeStruct((n_rows * _TOP_K,), jnp.float32),
            jax.ShapeDtypeStruct((n_rows * _TOP_K,), jnp.int32),
        ],
        mesh=mesh,
        scratch_types=[
            pltpu.VMEM((rpw, 64), jnp.float32),
            pltpu.VMEM((rpw * _TOP_K + 8,), jnp.float32),
            pltpu.VMEM((rpw * _TOP_K + 8,), jnp.int32),
        ],
        compiler_params=pltpu.CompilerParams(needs_layout_passes=False),
    )
    def k(logits_hbm, w_hbm, i_hbm, rows_v, ow_v, oi_v):
        wid = lax.axis_index("s") * nc + lax.axis_index("c")
        base = wid * rpw
        pltpu.sync_copy(logits_hbm.at[pl.ds(base, rpw)], rows_v)

        lane = lax.iota(jnp.int32, _L)
        shift8 = jnp.maximum(lane - 8, 0)
        low8 = lane < 8

        def _merge(a, ai, b, bi):
            ck = jnp.where(low8, a, _shuf(b, shift8))
            cv = jnp.where(low8, ai, _shuf(bi, shift8))
            return plsc.sort_key_val(ck, cv, descending=True)

        def row_body(r, _):
            s = []
            for j in range(4):
                v = rows_v[r, pl.ds(j * _L, _L)]
                s.append(plsc.sort_key_val(v, lane + j * _L, descending=True))
            k01, v01 = _merge(s[0][0], s[0][1], s[1][0], s[1][1])
            k23, v23 = _merge(s[2][0], s[2][1], s[3][0], s[3][1])
            fk, fv = _merge(k01, v01, k23, v23)
            e = jnp.where(low8, jnp.exp(fk - jnp.max(fk)), 0.0)
            s_vec = lax.broadcast_in_dim(jnp.sum(e), (_L,), ())
            wts = e / s_vec
            ow_v[pl.ds(r * _TOP_K, _L)] = wts
            oi_v[pl.ds(r * _TOP_K, _L)] = fv
            return 0

        lax.fori_loop(0, rpw, row_body, 0)
        pltpu.sync_copy(ow_v.at[pl.ds(0, rpw * _TOP_K)],
                        w_hbm.at[pl.ds(base * _TOP_K, rpw * _TOP_K)])
        pltpu.sync_copy(oi_v.at[pl.ds(0, rpw * _TOP_K)],
                        i_hbm.at[pl.ds(base * _TOP_K, rpw * _TOP_K)])

    return k(logits)


@jax.jit
def kernel(hidden_states, W):
    n_tokens = hidden_states.shape[0]
    logits_parts, rw_parts, idx_parts = [], [], []
    base = 0
    for sz in _CHUNKS:
        lg = _tc_logits(hidden_states, W, base, sz)
        rw_f, idx_f = _sc_topk(lg, sz)
        logits_parts.append(lg)
        rw_parts.append(rw_f.reshape(sz, _TOP_K))
        idx_parts.append(idx_f.reshape(sz, _TOP_K))
        base += sz
    logits = jnp.concatenate(logits_parts, axis=0)
    rw = jnp.concatenate(rw_parts, axis=0)
    idx = jnp.concatenate(idx_parts, axis=0)
    return rw, idx, logits


# speedup vs baseline: 1.0124x; 1.0124x over previous
import functools

import jax
import jax.numpy as jnp
from jax import lax
from jax.experimental import pallas as pl
from jax.experimental.pallas import tpu as pltpu
from jax.experimental.pallas import tpu_sc as plsc

_TOP_K = 8
_BT = 512
_CHUNKS = (4096, 4096, 4096, 4096)
_L = 16


def _matmul_body(x_ref, w_ref, logits_ref):
    logits_ref[...] = jax.lax.dot_general(
        x_ref[...], w_ref[...], (((1,), (1,)), ((), ())),
        preferred_element_type=jnp.float32,
    )


def _tc_logits(x, w, base_tokens, n_chunk_tokens):
    hidden_dim = x.shape[1]
    n_experts = w.shape[0]
    base = base_tokens // _BT
    return pl.pallas_call(
        _matmul_body,
        grid=(n_chunk_tokens // _BT,),
        in_specs=[
            pl.BlockSpec((_BT, hidden_dim), lambda i: (base + i, 0)),
            pl.BlockSpec((n_experts, hidden_dim), lambda i: (0, 0)),
        ],
        out_specs=pl.BlockSpec((_BT, n_experts), lambda i: (i, 0)),
        out_shape=jax.ShapeDtypeStruct((n_chunk_tokens, n_experts), jnp.float32),
    )(x, w)


def _sc_topk(logits, n_rows):
    nc, ns = 2, 16
    nw = nc * ns
    rpw = n_rows // nw

    gdn = lax.GatherDimensionNumbers(
        offset_dims=(), collapsed_slice_dims=(0,), start_index_map=(0,)
    )

    def _shuf(x, idx):
        return lax.gather(
            x, idx[:, None], gdn, (1,),
            mode=lax.GatherScatterMode.PROMISE_IN_BOUNDS,
        )

    mesh = plsc.VectorSubcoreMesh(
        core_axis_name="c", subcore_axis_name="s", num_cores=nc, num_subcores=ns
    )

    @functools.partial(
        pl.kernel,
        out_type=[
            jax.ShapeDtypeStruct((n_rows * _TOP_K,), jnp.float32),
            jax.ShapeDtypeStruct((n_rows * _TOP_K,), jnp.int32),
        ],
        mesh=mesh,
        scratch_types=[
            pltpu.VMEM((rpw, 64), jnp.float32),
            pltpu.VMEM((rpw * _TOP_K + 8,), jnp.float32),
            pltpu.VMEM((rpw * _TOP_K + 8,), jnp.int32),
        ],
        compiler_params=pltpu.CompilerParams(needs_layout_passes=False),
    )
    def k(logits_hbm, w_hbm, i_hbm, rows_v, ow_v, oi_v):
        wid = lax.axis_index("s") * nc + lax.axis_index("c")
        base = wid * rpw
        pltpu.sync_copy(logits_hbm.at[pl.ds(base, rpw)], rows_v)

        lane = lax.iota(jnp.int32, _L)
        shift8 = jnp.maximum(lane - 8, 0)
        low8 = lane < 8

        def _merge(a, ai, b, bi):
            ck = jnp.where(low8, a, _shuf(b, shift8))
            cv = jnp.where(low8, ai, _shuf(bi, shift8))
            return plsc.sort_key_val(ck, cv, descending=True)

        def row_body(r, _):
            s = []
            for j in range(4):
                v = rows_v[r, pl.ds(j * _L, _L)]
                s.append(plsc.sort_key_val(v, lane + j * _L, descending=True))
            k01, v01 = _merge(s[0][0], s[0][1], s[1][0], s[1][1])
            k23, v23 = _merge(s[2][0], s[2][1], s[3][0], s[3][1])
            fk, fv = _merge(k01, v01, k23, v23)
            e = jnp.where(low8, jnp.exp(fk - jnp.max(fk)), 0.0)
            s_vec = lax.broadcast_in_dim(jnp.sum(e), (_L,), ())
            wts = e / s_vec
            ow_v[pl.ds(r * _TOP_K, _L)] = wts
            oi_v[pl.ds(r * _TOP_K, _L)] = fv
            return 0

        lax.fori_loop(0, rpw, row_body, 0)
        pltpu.sync_copy(ow_v.at[pl.ds(0, rpw * _TOP_K)],
                        w_hbm.at[pl.ds(base * _TOP_K, rpw * _TOP_K)])
        pltpu.sync_copy(oi_v.at[pl.ds(0, rpw * _TOP_K)],
                        i_hbm.at[pl.ds(base * _TOP_K, rpw * _TOP_K)])

    return k(logits)


@jax.jit
def kernel(hidden_states, W):
    n_tokens = hidden_states.shape[0]
    logits_parts, rw_parts, idx_parts = [], [], []
    base = 0
    for sz in _CHUNKS:
        lg = _tc_logits(hidden_states, W, base, sz)
        rw_f, idx_f = _sc_topk(lg, sz)
        logits_parts.append(lg)
        rw_parts.append(rw_f.reshape(sz, _TOP_K))
        idx_parts.append(idx_f.reshape(sz, _TOP_K))
        base += sz
    logits = jnp.concatenate(logits_parts, axis=0)
    rw = jnp.concatenate(rw_parts, axis=0)
    idx = jnp.concatenate(idx_parts, axis=0)
    return rw, idx, logits
